# baseline (speedup 1.0000x reference)
import functools

import jax
import jax.numpy as jnp
from jax import lax
from jax.experimental import pallas as pl
from jax.experimental.pallas import tpu as pltpu
from jax.experimental.pallas import tpu_sc as plsc

E_DIM = 512
N_REL = 600
NC = 2
NS = 16
NW = NC * NS
LANES = 16
P_T = 512
N_T = 16384
CH = 8
S = 3

_CP = P_T // NW
_CN = N_T // NW
_NPC = _CP // CH
_NNC = _CN // CH
_TOTAL = _NPC + _NNC


def _box_body(rhb, rhw, rhs, rtb, rtw, rts, out):
    def half(base_ref, width_ref, scale_ref):
        w = width_ref[...]
        step2 = jnp.abs(w) + 1e-8
        norm_volume = jnp.exp(jnp.mean(jnp.log(step2), axis=1, keepdims=True))
        wn = w / norm_volume
        sc = scale_ref[...]
        s = jnp.where(sc > 0, sc, jnp.exp(sc) - 1.0) + 1.0
        d = wn * s
        b = base_ref[...]
        c1 = b + d
        c2 = b - d
        return jnp.maximum(c1, c2), jnp.minimum(c1, c2)

    hu, hl = half(rhb, rhw, rhs)
    tu, tl = half(rtb, rtw, rts)
    out[:, 0 * E_DIM:1 * E_DIM] = hu
    out[:, 1 * E_DIM:2 * E_DIM] = hl
    out[:, 2 * E_DIM:3 * E_DIM] = tu
    out[:, 3 * E_DIM:4 * E_DIM] = tl


def _box_tables(rhb, rhw, rhs, rtb, rtw, rts):
    rows = 120
    grid = N_REL // rows
    full = lambda i: (i, 0)
    return pl.pallas_call(
        _box_body,
        grid=(grid,),
        in_specs=[
            pl.BlockSpec((rows, E_DIM), full),
            pl.BlockSpec((rows, E_DIM), full),
            pl.BlockSpec((rows, 1), full),
            pl.BlockSpec((rows, E_DIM), full),
            pl.BlockSpec((rows, E_DIM), full),
            pl.BlockSpec((rows, 1), full),
        ],
        out_specs=pl.BlockSpec((rows, 4 * E_DIM), full),
        out_shape=jax.ShapeDtypeStruct((N_REL, 4 * E_DIM), jnp.float32),
    )(rhb, rhw, rhs, rtb, rtw, rts)


def _mesh():
    return plsc.VectorSubcoreMesh(
        core_axis_name="c", subcore_axis_name="s", num_cores=NC, num_subcores=NS
    )


@functools.lru_cache(maxsize=None)
def _sc_kernel():
    W2 = 2 * E_DIM
    W4 = 4 * E_DIM

    scratch = (
        [pltpu.VMEM((_CP,), jnp.int32)] * 3
        + [pltpu.VMEM((_CN,), jnp.int32)] * 3
        + [pltpu.VMEM((CH, W2), jnp.float32)] * (2 * S)
        + [pltpu.VMEM((CH, W4), jnp.float32)] * S
        + [pltpu.SemaphoreType.DMA] * (4 * S)
    )

    @functools.partial(
        pl.kernel,
        mesh=_mesh(),
        out_type=(
            jax.ShapeDtypeStruct((P_T, W2), jnp.float32),
            jax.ShapeDtypeStruct((N_T, W2), jnp.float32),
            jax.ShapeDtypeStruct((P_T, W4), jnp.float32),
            jax.ShapeDtypeStruct((N_T, W4), jnp.float32),
        ),
        scratch_types=scratch,
    )
    def k(hp_hbm, tp_hbm, rp_hbm, hn_hbm, tn_hbm, rn_hbm, ent2_hbm, boxes_hbm,
          pe_hbm, ne_hbm, pr_hbm, nr_hbm, *sc):
        hidx_p, tidx_p, ridx_p, hidx_n, tidx_n, ridx_n = sc[0:6]
        hb = sc[6:6 + S]
        tb = sc[6 + S:6 + 2 * S]
        rb = sc[6 + 2 * S:6 + 3 * S]
        egs = sc[6 + 3 * S:6 + 4 * S]
        ews = sc[6 + 4 * S:6 + 5 * S]
        rgs = sc[6 + 5 * S:6 + 6 * S]
        rws = sc[6 + 6 * S:6 + 7 * S]

        wid = lax.axis_index("s") * NC + lax.axis_index("c")
        pltpu.sync_copy(hp_hbm.at[pl.ds(wid * _CP, _CP)], hidx_p)
        pltpu.sync_copy(tp_hbm.at[pl.ds(wid * _CP, _CP)], tidx_p)
        pltpu.sync_copy(rp_hbm.at[pl.ds(wid * _CP, _CP)], ridx_p)
        pltpu.sync_copy(hn_hbm.at[pl.ds(wid * _CN, _CN)], hidx_n)
        pltpu.sync_copy(tn_hbm.at[pl.ds(wid * _CN, _CN)], tidx_n)
        pltpu.sync_copy(rn_hbm.at[pl.ds(wid * _CN, _CN)], ridx_n)

        def fire_pos(j, s):
            off = j * CH
            pltpu.async_copy(ent2_hbm.at[hidx_p.at[pl.ds(off, CH)]], hb[s], egs[s])
            pltpu.async_copy(ent2_hbm.at[tidx_p.at[pl.ds(off, CH)]], tb[s], egs[s])
            pltpu.async_copy(boxes_hbm.at[ridx_p.at[pl.ds(off, CH)]], rb[s], rgs[s])

        def fire_neg(jj, s):
            off = jj * CH
            pltpu.async_copy(ent2_hbm.at[hidx_n.at[pl.ds(off, CH)]], hb[s], egs[s])
            pltpu.async_copy(ent2_hbm.at[tidx_n.at[pl.ds(off, CH)]], tb[s], egs[s])
            pltpu.async_copy(boxes_hbm.at[ridx_n.at[pl.ds(off, CH)]], rb[s], rgs[s])

        def wait_ent_g(s):
            pltpu.make_async_copy(ne_hbm.at[pl.ds(0, CH)], hb[s], egs[s]).wait()
            pltpu.make_async_copy(ne_hbm.at[pl.ds(0, CH)], tb[s], egs[s]).wait()

        def wait_rel_g(s):
            pltpu.make_async_copy(nr_hbm.at[pl.ds(0, CH)], rb[s], rgs[s]).wait()

        def compute(s):
            def row(i, _):
                def vec(kk, _):
                    lo = kk * LANES
                    hi = E_DIM + lo
                    a = hb[s][i, pl.ds(lo, LANES)]
                    bv = tb[s][i, pl.ds(hi, LANES)]
                    cv = tb[s][i, pl.ds(lo, LANES)]
                    dv = hb[s][i, pl.ds(hi, LANES)]
                    hb[s][i, pl.ds(lo, LANES)] = a + bv
                    hb[s][i, pl.ds(hi, LANES)] = cv + dv
                    return 0

                lax.fori_loop(0, E_DIM // LANES, vec, 0, unroll=4)
                return 0

            lax.fori_loop(0, CH, row, 0)

        def wr_pos(j, s):
            off = j * CH
            pltpu.async_copy(hb[s], pe_hbm.at[pl.ds(wid * _CP + off, CH)], ews[s])
            pltpu.async_copy(rb[s], pr_hbm.at[pl.ds(wid * _CP + off, CH)], rws[s])

        def wr_neg(jj, s):
            off = jj * CH
            pltpu.async_copy(hb[s], ne_hbm.at[pl.ds(wid * _CN + off, CH)], ews[s])
            pltpu.async_copy(rb[s], nr_hbm.at[pl.ds(wid * _CN + off, CH)], rws[s])

        def drain_w(s):
            pltpu.make_async_copy(hb[s], ne_hbm.at[pl.ds(0, CH)], ews[s]).wait()
            pltpu.make_async_copy(rb[s], nr_hbm.at[pl.ds(0, CH)], rws[s]).wait()

        fire_pos(0, 0)
        fire_pos(1, 1)
        for j in (0, 1):
            wait_ent_g(j)
            compute(j)
            wait_rel_g(j)
            wr_pos(j, j)
            if j == 0:
                fire_neg(0, 2)
            else:
                drain_w(0)
                fire_neg(1, 0)
        wait_ent_g(2)
        compute(2)
        wait_rel_g(2)
        wr_neg(0, 2)
        drain_w(1)
        fire_neg(2, 1)

        def group(g, _):
            for b in range(S):
                jj = 3 * g + b - 2
                s = b
                wait_ent_g(s)
                compute(s)
                wait_rel_g(s)
                wr_neg(jj, s)
                nxt = (b + 2) % S

                @pl.when(jj + 2 < _NNC)
                def _():
                    drain_w(nxt)
                    fire_neg(jj + 2, nxt)

                _ = _
            return 0

        lax.fori_loop(1, _TOTAL // S, group, 0)
        drain_w(0)
        drain_w(1)
        drain_w(2)

    return k


def kernel(positives, negatives, r_head_base_points, r_head_widths,
           r_head_size_scales, r_tail_base_points, r_tail_widths,
           r_tail_size_scales, entity_bases, entity_bumps):
    boxes = _box_tables(r_head_base_points, r_head_widths, r_head_size_scales,
                        r_tail_base_points, r_tail_widths, r_tail_size_scales)
    ent2 = jnp.concatenate([entity_bases, entity_bumps], axis=1)

    def ids(tuples, col):
        return tuples[:, col, :].reshape(-1).astype(jnp.int32)

    hp, rp, tp = ids(positives, 0), ids(positives, 1), ids(positives, 2)
    hn, rn, tn = ids(negatives, 0), ids(negatives, 1), ids(negatives, 2)

    pe, ne, pr, nr = _sc_kernel()(hp, tp, rp, hn, tn, rn, ent2, boxes)

    p_ent = pe.reshape(1, P_T, 2, E_DIM)
    n_ent = ne.reshape(32, N_T // 32, 2, E_DIM)
    p_rel = pr.reshape(1, P_T, 2, 2, E_DIM)
    n_rel = nr.reshape(32, N_T // 32, 2, 2, E_DIM)
    return (p_ent, p_rel, n_ent, n_rel)


# speedup vs baseline: 2.2147x; 2.2147x over previous
import functools

import jax
import jax.numpy as jnp
from jax import lax
from jax.experimental import pallas as pl
from jax.experimental.pallas import tpu as pltpu
from jax.experimental.pallas import tpu_sc as plsc

E_DIM = 512
N_REL = 600
BATCH = 512
NNEG = 32
NC = 2
NS = 16
NW = NC * NS
LANES = 16
P_T = BATCH
N_T = NNEG * BATCH
CH = 8
S = 3

_CP = P_T // NW
_CN = N_T // NW
_NPC = _CP // CH
_NNC = _CN // CH
_TOTAL = _NPC + _NNC


def _box_body(rhb, rhw, rhs, rtb, rtw, rts, out):
    def half(base_ref, width_ref, scale_ref):
        w = width_ref[...]
        step2 = jnp.abs(w) + 1e-8
        norm_volume = jnp.exp(jnp.mean(jnp.log(step2), axis=1, keepdims=True))
        wn = w / norm_volume
        sc = scale_ref[...]
        s = jnp.where(sc > 0, sc, jnp.exp(sc) - 1.0) + 1.0
        d = wn * s
        b = base_ref[...]
        c1 = b + d
        c2 = b - d
        return jnp.maximum(c1, c2), jnp.minimum(c1, c2)

    hu, hl = half(rhb, rhw, rhs)
    tu, tl = half(rtb, rtw, rts)
    out[:, 0 * E_DIM:1 * E_DIM] = hu
    out[:, 1 * E_DIM:2 * E_DIM] = hl
    out[:, 2 * E_DIM:3 * E_DIM] = tu
    out[:, 3 * E_DIM:4 * E_DIM] = tl


def _box_tables(rhb, rhw, rhs, rtb, rtw, rts):
    rows = 120
    grid = N_REL // rows
    full = lambda i: (i, 0)
    return pl.pallas_call(
        _box_body,
        grid=(grid,),
        in_specs=[
            pl.BlockSpec((rows, E_DIM), full),
            pl.BlockSpec((rows, E_DIM), full),
            pl.BlockSpec((rows, 1), full),
            pl.BlockSpec((rows, E_DIM), full),
            pl.BlockSpec((rows, E_DIM), full),
            pl.BlockSpec((rows, 1), full),
        ],
        out_specs=pl.BlockSpec((rows, 4 * E_DIM), full),
        out_shape=jax.ShapeDtypeStruct((N_REL, 4 * E_DIM), jnp.float32),
    )(rhb, rhw, rhs, rtb, rtw, rts)


def _mesh():
    return plsc.VectorSubcoreMesh(
        core_axis_name="c", subcore_axis_name="s", num_cores=NC, num_subcores=NS
    )


@functools.lru_cache(maxsize=None)
def _sc_kernel():
    scratch = (
        [pltpu.VMEM((_CP,), jnp.int32)] * 3
        + [pltpu.VMEM((_CN,), jnp.int32)] * 3
        + [pltpu.VMEM((CH, 2, E_DIM), jnp.float32)] * (2 * S)
        + [pltpu.VMEM((CH, 2, E_DIM), jnp.float32)] * S
        + [pltpu.VMEM((CH, 2, 2, E_DIM), jnp.float32)] * S
        + [pltpu.SemaphoreType.DMA] * (4 * S)
    )

    @functools.partial(
        pl.kernel,
        mesh=_mesh(),
        out_type=(
            jax.ShapeDtypeStruct((1, P_T, 2, E_DIM), jnp.float32),
            jax.ShapeDtypeStruct((NNEG, BATCH, 2, E_DIM), jnp.float32),
            jax.ShapeDtypeStruct((1, P_T, 2, 2, E_DIM), jnp.float32),
            jax.ShapeDtypeStruct((NNEG, BATCH, 2, 2, E_DIM), jnp.float32),
        ),
        scratch_types=scratch,
    )
    def k(hp_hbm, tp_hbm, rp_hbm, hn_hbm, tn_hbm, rn_hbm, ent2_hbm, boxes_hbm,
          pe_hbm, ne_hbm, pr_hbm, nr_hbm, *sc):
        hidx_p, tidx_p, ridx_p, hidx_n, tidx_n, ridx_n = sc[0:6]
        hb = sc[6:6 + S]
        tb = sc[6 + S:6 + 2 * S]
        eb = sc[6 + 2 * S:6 + 3 * S]
        rb = sc[6 + 3 * S:6 + 4 * S]
        egs = sc[6 + 4 * S:6 + 5 * S]
        ews = sc[6 + 5 * S:6 + 6 * S]
        rgs = sc[6 + 6 * S:6 + 7 * S]
        rws = sc[6 + 7 * S:6 + 8 * S]

        wid = lax.axis_index("s") * NC + lax.axis_index("c")
        pltpu.sync_copy(hp_hbm.at[pl.ds(wid * _CP, _CP)], hidx_p)
        pltpu.sync_copy(tp_hbm.at[pl.ds(wid * _CP, _CP)], tidx_p)
        pltpu.sync_copy(rp_hbm.at[pl.ds(wid * _CP, _CP)], ridx_p)
        pltpu.sync_copy(hn_hbm.at[pl.ds(wid * _CN, _CN)], hidx_n)
        pltpu.sync_copy(tn_hbm.at[pl.ds(wid * _CN, _CN)], tidx_n)
        pltpu.sync_copy(rn_hbm.at[pl.ds(wid * _CN, _CN)], ridx_n)

        def fire_pos(j, s):
            off = j * CH
            pltpu.async_copy(ent2_hbm.at[hidx_p.at[pl.ds(off, CH)]], hb[s], egs[s])
            pltpu.async_copy(ent2_hbm.at[tidx_p.at[pl.ds(off, CH)]], tb[s], egs[s])
            pltpu.async_copy(boxes_hbm.at[ridx_p.at[pl.ds(off, CH)]], rb[s], rgs[s])

        def fire_neg(jj, s):
            off = jj * CH
            pltpu.async_copy(ent2_hbm.at[hidx_n.at[pl.ds(off, CH)]], hb[s], egs[s])
            pltpu.async_copy(ent2_hbm.at[tidx_n.at[pl.ds(off, CH)]], tb[s], egs[s])
            pltpu.async_copy(boxes_hbm.at[ridx_n.at[pl.ds(off, CH)]], rb[s], rgs[s])

        def wait_ent_g(s):
            pltpu.make_async_copy(ne_hbm.at[0, pl.ds(0, CH)], hb[s], egs[s]).wait()
            pltpu.make_async_copy(ne_hbm.at[0, pl.ds(0, CH)], tb[s], egs[s]).wait()

        def wait_rel_g(s):
            pltpu.make_async_copy(nr_hbm.at[0, pl.ds(0, CH)], rb[s], rgs[s]).wait()

        def compute(s):
            def row(i, _):
                def vec(kk, _):
                    sl = pl.ds(kk * LANES, LANES)
                    eb[s][i, 0, sl] = hb[s][i, 0, sl] + tb[s][i, 1, sl]
                    eb[s][i, 1, sl] = tb[s][i, 0, sl] + hb[s][i, 1, sl]
                    return 0

                lax.fori_loop(0, E_DIM // LANES, vec, 0, unroll=4)
                return 0

            lax.fori_loop(0, CH, row, 0)

        def wr_pos(j, s):
            off = wid * _CP + j * CH
            pltpu.async_copy(eb[s], pe_hbm.at[0, pl.ds(off, CH)], ews[s])
            pltpu.async_copy(rb[s], pr_hbm.at[0, pl.ds(off, CH)], rws[s])

        def wr_neg(jj, s):
            off = jj * CH
            pltpu.async_copy(eb[s], ne_hbm.at[wid, pl.ds(off, CH)], ews[s])
            pltpu.async_copy(rb[s], nr_hbm.at[wid, pl.ds(off, CH)], rws[s])

        def drain_w(s):
            pltpu.make_async_copy(eb[s], ne_hbm.at[0, pl.ds(0, CH)], ews[s]).wait()
            pltpu.make_async_copy(rb[s], nr_hbm.at[0, pl.ds(0, CH)], rws[s]).wait()

        fire_pos(0, 0)
        fire_pos(1, 1)
        for j in (0, 1):
            wait_ent_g(j)
            compute(j)
            wait_rel_g(j)
            wr_pos(j, j)
            if j == 0:
                fire_neg(0, 2)
            else:
                drain_w(0)
                fire_neg(1, 0)
        wait_ent_g(2)
        compute(2)
        wait_rel_g(2)
        wr_neg(0, 2)
        drain_w(1)
        fire_neg(2, 1)

        def group(g, _):
            for b in range(S):
                jj = 3 * g + b - 2
                s = b
                wait_ent_g(s)
                compute(s)
                wait_rel_g(s)
                wr_neg(jj, s)
                nxt = (b + 2) % S

                @pl.when(jj + 2 < _NNC)
                def _():
                    drain_w(nxt)
                    fire_neg(jj + 2, nxt)

                _ = _
            return 0

        lax.fori_loop(1, _TOTAL // S, group, 0)
        drain_w(0)
        drain_w(1)
        drain_w(2)

    return k


def kernel(positives, negatives, r_head_base_points, r_head_widths,
           r_head_size_scales, r_tail_base_points, r_tail_widths,
           r_tail_size_scales, entity_bases, entity_bumps):
    boxes = _box_tables(r_head_base_points, r_head_widths, r_head_size_scales,
                        r_tail_base_points, r_tail_widths, r_tail_size_scales)
    boxes = boxes.reshape(N_REL, 2, 2, E_DIM)
    ent2 = jnp.concatenate([entity_bases, entity_bumps], axis=1)
    ent2 = ent2.reshape(-1, 2, E_DIM)

    def ids(tuples, col):
        return tuples[:, col, :].reshape(-1).astype(jnp.int32)

    hp, rp, tp = ids(positives, 0), ids(positives, 1), ids(positives, 2)
    hn, rn, tn = ids(negatives, 0), ids(negatives, 1), ids(negatives, 2)

    p_ent, n_ent, p_rel, n_rel = _sc_kernel()(hp, tp, rp, hn, tn, rn, ent2, boxes)
    return (p_ent, p_rel, n_ent, n_rel)
